# Initial kernel scaffold; baseline (speedup 1.0000x reference)
import jax
import jax.numpy as jnp
from jax.experimental import pallas as pl


def kernel(tokens, embedding_table):
    raise NotImplementedError("write your pallas kernel here")


# speedup vs baseline: 1.0882x; 1.0882x over previous
import functools

import jax
import jax.numpy as jnp
from jax import lax
from jax.experimental import pallas as pl
from jax.experimental.pallas import tpu as pltpu
from jax.experimental.pallas import tpu_sc as plsc

VOCAB = 100000
D = 1024
B = 4
S = 4096
N_TOK = B * S
SCALE = 32.0

NC = 2
NS = 16
LANES = 16
NW = NC * NS
TPW = N_TOK // NW
CHUNK = 32
NCHUNK = TPW // CHUNK

_mesh = plsc.VectorSubcoreMesh(
    core_axis_name="c", subcore_axis_name="s", num_cores=NC, num_subcores=NS
)


@functools.partial(
    pl.kernel,
    out_type=jax.ShapeDtypeStruct((N_TOK, D), jnp.float32),
    mesh=_mesh,
    scratch_types=[
        pltpu.VMEM((TPW,), jnp.int32),
        pltpu.VMEM((CHUNK, D), jnp.float32),
        pltpu.SemaphoreType.DMA,
    ],
)
def _embed(tokens_hbm, table_hbm, out_hbm, idx_v, buf, gsem):
    wid = lax.axis_index("s") * NC + lax.axis_index("c")
    base = wid * TPW
    pltpu.sync_copy(tokens_hbm.at[pl.ds(base, TPW)], idx_v)

    def scale_chunk(buf):
        def row_body(r, _):
            row = buf.at[r]
            for j in range(D // LANES):
                sl = pl.ds(j * LANES, LANES)
                row[sl] = row[sl] * SCALE
            return _

        lax.fori_loop(0, CHUNK, row_body, 0)

    for k in range(NCHUNK):
        idx = idx_v.at[pl.ds(k * CHUNK, CHUNK)]
        pltpu.async_copy(table_hbm.at[idx], buf, gsem).wait()
        scale_chunk(buf)
        pltpu.sync_copy(buf, out_hbm.at[pl.ds(base + k * CHUNK, CHUNK)])


def kernel(tokens, embedding_table):
    flat = tokens.reshape(N_TOK).astype(jnp.int32)
    out = _embed(flat, embedding_table)
    return out.reshape(B, S, D)
